# Initial kernel scaffold; baseline (speedup 1.0000x reference)
import jax
import jax.numpy as jnp
from jax.experimental import pallas as pl


def kernel(x, edge_index, W, b):
    raise NotImplementedError("write your pallas kernel here")


# speedup vs baseline: 18.1986x; 18.1986x over previous
import functools
import jax
import jax.numpy as jnp
from jax import lax
from jax.experimental import pallas as pl
from jax.experimental.pallas import tpu as pltpu
from jax.experimental.pallas import tpu_sc as plsc

N_NODES = 10000
N_PAD = 10240
N_EDGES = 320000
D = 128

NC, NS = 2, 16
NW = NC * NS
E_PER_TILE = N_EDGES // NW
CH = 80
N_CHUNKS = E_PER_TILE // CH
ROWS_PT = N_PAD // NS

_mesh = plsc.VectorSubcoreMesh(
    core_axis_name="c", subcore_axis_name="s", num_cores=NC, num_subcores=NS)

_f32 = jnp.float32


def _zero_vmem_2d(ref, rows, cols):
  z = jnp.zeros((16,), _f32)
  per_row = cols // 16

  def body(i, _):
    ref[i // per_row, pl.ds((i % per_row) * 16, 16)] = z
    return 0

  lax.fori_loop(0, rows * per_row, body, 0)


def _zero_vmem_1d(ref, n):
  z = jnp.zeros((16,), _f32)

  def body(i, _):
    ref[pl.ds(i * 16, 16)] = z
    return 0

  lax.fori_loop(0, n // 16, body, 0)


@functools.partial(
    pl.kernel,
    out_type=jax.ShapeDtypeStruct((NC, N_PAD), _f32),
    mesh=_mesh,
    scratch_types=[
        pltpu.VMEM((CH,), jnp.int32),
        pltpu.VMEM((CH,), _f32),
        pltpu.VMEM((ROWS_PT,), _f32),
        pltpu.VMEM_SHARED((N_PAD,), _f32),
    ],
)
def _deg_kernel(dst_hbm, deg_out, idx_v, ones_v, zbuf_v, acc_sh):
  c = lax.axis_index("c")
  s = lax.axis_index("s")
  wid = c * NS + s

  _zero_vmem_1d(zbuf_v, ROWS_PT)
  one = jnp.ones((16,), _f32)
  for j in range(CH // 16):
    ones_v[pl.ds(j * 16, 16)] = one
  pltpu.sync_copy(zbuf_v, acc_sh.at[pl.ds(s * ROWS_PT, ROWS_PT)])
  plsc.subcore_barrier()

  def body(i, _):
    base = wid * E_PER_TILE + i * CH
    pltpu.sync_copy(dst_hbm.at[pl.ds(base, CH)], idx_v)
    pltpu.sync_copy(ones_v, acc_sh.at[idx_v], add=True)
    return 0

  lax.fori_loop(0, N_CHUNKS, body, 0)
  plsc.subcore_barrier()
  pltpu.sync_copy(acc_sh.at[pl.ds(s * ROWS_PT, ROWS_PT)],
                  deg_out.at[c, pl.ds(s * ROWS_PT, ROWS_PT)])


@functools.partial(
    pl.kernel,
    out_type=jax.ShapeDtypeStruct((NC, N_PAD, D), _f32),
    mesh=_mesh,
    scratch_types=[
        pltpu.VMEM((CH,), jnp.int32),
        pltpu.VMEM((CH,), jnp.int32),
        pltpu.VMEM((CH, D), _f32),
        pltpu.VMEM_SHARED((N_PAD, D), _f32),
        pltpu.SemaphoreType.DMA,
    ],
)
def _edge_kernel(src_hbm, dst_hbm, g_hbm, part_out, src_v, dst_v, rows_v,
                 acc_sh, sem):
  c = lax.axis_index("c")
  s = lax.axis_index("s")
  wid = c * NS + s

  _zero_vmem_2d(rows_v, CH, D)
  for k in range(ROWS_PT // CH):
    pltpu.sync_copy(rows_v, acc_sh.at[pl.ds(s * ROWS_PT + k * CH, CH), :])
  plsc.subcore_barrier()

  def body(i, _):
    base = wid * E_PER_TILE + i * CH
    pltpu.sync_copy(src_hbm.at[pl.ds(base, CH)], src_v)
    pltpu.sync_copy(dst_hbm.at[pl.ds(base, CH)], dst_v)
    pltpu.async_copy(g_hbm.at[src_v], rows_v, sem).wait()
    pltpu.sync_copy(rows_v, acc_sh.at[dst_v], add=True)
    return 0

  lax.fori_loop(0, N_CHUNKS, body, 0)
  plsc.subcore_barrier()
  pltpu.sync_copy(acc_sh.at[pl.ds(s * ROWS_PT, ROWS_PT), :],
                  part_out.at[c, pl.ds(s * ROWS_PT, ROWS_PT), :])


_RB = 2048


def _prep_body(x_ref, w_ref, degp_ref, g_ref):
  h = jnp.dot(x_ref[...], w_ref[...], preferred_element_type=_f32)
  deg = degp_ref[0, :] + degp_ref[1, :] + 1.0
  dinv = lax.rsqrt(deg)
  g_ref[...] = h * dinv[:, None]


def _combine_body(p_ref, g_ref, degp_ref, b_ref, out_ref):
  deg = degp_ref[0, :] + degp_ref[1, :] + 1.0
  dinv = lax.rsqrt(deg)
  acc = p_ref[0] + p_ref[1] + g_ref[...]
  out_ref[...] = acc * dinv[:, None] + b_ref[0, :]


def kernel(x, edge_index, W, b):
  ei = edge_index.astype(jnp.int32)
  src, dst = ei[0], ei[1]
  xp = jnp.zeros((N_PAD, D), _f32).at[:N_NODES].set(x)

  deg_part = _deg_kernel(dst)

  g = pl.pallas_call(
      _prep_body,
      grid=(N_PAD // _RB,),
      in_specs=[
          pl.BlockSpec((_RB, D), lambda i: (i, 0)),
          pl.BlockSpec((D, D), lambda i: (0, 0)),
          pl.BlockSpec((NC, _RB), lambda i: (0, i)),
      ],
      out_specs=pl.BlockSpec((_RB, D), lambda i: (i, 0)),
      out_shape=jax.ShapeDtypeStruct((N_PAD, D), _f32),
  )(xp, W, deg_part)

  part = _edge_kernel(src, dst, g)

  out = pl.pallas_call(
      _combine_body,
      grid=(N_PAD // _RB,),
      in_specs=[
          pl.BlockSpec((NC, _RB, D), lambda i: (0, i, 0)),
          pl.BlockSpec((_RB, D), lambda i: (i, 0)),
          pl.BlockSpec((NC, _RB), lambda i: (0, i)),
          pl.BlockSpec((1, D), lambda i: (0, 0)),
      ],
      out_specs=pl.BlockSpec((_RB, D), lambda i: (i, 0)),
      out_shape=jax.ShapeDtypeStruct((N_PAD, D), _f32),
  )(part, g, deg_part, b.reshape(1, D))

  return out[:N_NODES]


# speedup vs baseline: 38.4431x; 2.1124x over previous
import functools
import jax
import jax.numpy as jnp
from jax import lax
from jax.experimental import pallas as pl
from jax.experimental.pallas import tpu as pltpu
from jax.experimental.pallas import tpu_sc as plsc

N_NODES = 10000
N_PAD = 10240
N_EDGES = 320000
D = 128
DH = D // 2

NC, NS = 2, 16
NW = NC * NS
E_PER_TILE = N_EDGES // NW
CH = 80
N_CHUNKS = E_PER_TILE // CH
ROWS_PT = N_PAD // NS
NGRP = 15

_mesh = plsc.VectorSubcoreMesh(
    core_axis_name="c", subcore_axis_name="s", num_cores=NC, num_subcores=NS)

_f32 = jnp.float32


def _zero_vmem_2d(ref, rows, cols):
  z = jnp.zeros((16,), _f32)
  per_row = cols // 16

  def body(i, _):
    ref[i // per_row, pl.ds((i % per_row) * 16, 16)] = z
    return 0

  lax.fori_loop(0, rows * per_row, body, 0)


@functools.partial(
    pl.kernel,
    out_type=jax.ShapeDtypeStruct((NC, N_PAD), _f32),
    mesh=_mesh,
    scratch_types=[
        pltpu.VMEM((N_CHUNKS, CH), jnp.int32),
        pltpu.VMEM((CH,), _f32),
        pltpu.VMEM((ROWS_PT,), _f32),
        pltpu.VMEM_SHARED((N_PAD,), _f32),
        pltpu.SemaphoreType.DMA,
    ],
)
def _deg_kernel(dst_hbm, deg_out, idx_v, ones_v, zbuf_v, acc_sh, sem):
  c = lax.axis_index("c")
  s = lax.axis_index("s")
  wid = c * NS + s

  z = jnp.zeros((16,), _f32)
  for j in range(ROWS_PT // 16):
    zbuf_v[pl.ds(j * 16, 16)] = z
  one = jnp.ones((16,), _f32)
  for j in range(CH // 16):
    ones_v[pl.ds(j * 16, 16)] = one
  pltpu.sync_copy(zbuf_v, acc_sh.at[pl.ds(s * ROWS_PT, ROWS_PT)])
  pltpu.sync_copy(dst_hbm.at[wid], idx_v)
  plsc.subcore_barrier()

  W = 16

  def body(t, _):
    pltpu.async_copy(ones_v, acc_sh.at[idx_v.at[t]], sem, add=True)

    @pl.when(t >= W)
    def _():
      pltpu.make_async_copy(ones_v, acc_sh.at[idx_v.at[0]], sem).wait()

    return 0

  lax.fori_loop(0, N_CHUNKS, body, 0)

  def drain(t, _):
    pltpu.make_async_copy(ones_v, acc_sh.at[idx_v.at[0]], sem).wait()
    return 0

  lax.fori_loop(0, W, drain, 0)
  plsc.subcore_barrier()
  pltpu.sync_copy(acc_sh.at[pl.ds(s * ROWS_PT, ROWS_PT)],
                  deg_out.at[c, pl.ds(s * ROWS_PT, ROWS_PT)])


@functools.partial(
    pl.kernel,
    out_type=jax.ShapeDtypeStruct((NC, 2, N_PAD, DH), _f32),
    mesh=_mesh,
    scratch_types=[
        pltpu.VMEM((N_CHUNKS, CH), jnp.int32),
        pltpu.VMEM((N_CHUNKS, CH), jnp.int32),
        [pltpu.VMEM((CH, DH), _f32)] * 8,
        pltpu.VMEM_SHARED((N_PAD, DH), _f32),
        [pltpu.SemaphoreType.DMA] * 8,
    ],
    compiler_params=pltpu.CompilerParams(use_tc_tiling_on_sc=False),
)
def _edge_kernel(src_hbm, dst_hbm, ga_hbm, gb_hbm, part_out, src_v, dst_v,
                 bufs, acc_sh, sems):
  c = lax.axis_index("c")
  s = lax.axis_index("s")
  wid = c * NS + s

  pltpu.sync_copy(src_hbm.at[wid], src_v)
  pltpu.sync_copy(dst_hbm.at[wid], dst_v)
  _zero_vmem_2d(bufs[0], CH, DH)
  for k in range(ROWS_PT // CH):
    pltpu.sync_copy(bufs[0], acc_sh.at[pl.ds(s * ROWS_PT + k * CH, CH), :])
  plsc.subcore_barrier()

  for p, g_hbm in enumerate((ga_hbm, gb_hbm)):

    def fire_gather(t, k):
      pltpu.async_copy(g_hbm.at[src_v.at[t]], bufs[k], sems[k])

    def wait_gather(k):
      pltpu.make_async_copy(g_hbm.at[src_v.at[0]], bufs[k], sems[k]).wait()

    def fire_scatter(t, k):
      pltpu.async_copy(bufs[k], acc_sh.at[dst_v.at[t]], sems[k], add=True)

    def wait_scatter(k):
      pltpu.make_async_copy(bufs[k], acc_sh.at[dst_v.at[0]], sems[k]).wait()

    for k in range(4):
      fire_gather(k, k)

    def body(i, _):
      t0 = i * 8

      @pl.when(i > 0)
      def _():
        for k in range(4):
          wait_scatter(4 + k)
      for k in range(4):
        fire_gather(t0 + 4 + k, 4 + k)
      for k in range(4):
        wait_gather(k)
        fire_scatter(t0 + k, k)

      for k in range(4):
        wait_scatter(k)
      for k in range(4):
        fire_gather(t0 + 8 + k, k)
      for k in range(4):
        wait_gather(4 + k)
        fire_scatter(t0 + 4 + k, 4 + k)
      return 0

    lax.fori_loop(0, NGRP, body, 0)

    wait_scatter(4)
    fire_gather(N_CHUNKS - 1, 4)
    for k in range(4):
      wait_gather(k)
      fire_scatter(120 + k, k)
    wait_gather(4)
    fire_scatter(N_CHUNKS - 1, 4)
    for k in range(4):
      wait_scatter(k)
    for k in range(1, 4):
      wait_scatter(4 + k)
    wait_scatter(4)

    plsc.subcore_barrier()
    pltpu.sync_copy(acc_sh.at[pl.ds(s * ROWS_PT, ROWS_PT), :],
                    part_out.at[c, p, pl.ds(s * ROWS_PT, ROWS_PT), :])
    if p == 0:
      _zero_vmem_2d(bufs[0], CH, DH)
      for k in range(ROWS_PT // CH):
        pltpu.sync_copy(bufs[0], acc_sh.at[pl.ds(s * ROWS_PT + k * CH, CH), :])
      plsc.subcore_barrier()


_RB = 2048


def _prep_body(x_ref, w_ref, degp_ref, ga_ref, gb_ref):
  h = jnp.dot(x_ref[...], w_ref[...], preferred_element_type=_f32)
  deg = degp_ref[0, :] + degp_ref[1, :] + 1.0
  dinv = lax.rsqrt(deg)
  g = h * dinv[:, None]
  ga_ref[...] = g[:, :DH]
  gb_ref[...] = g[:, DH:]


def _combine_body(p_ref, ga_ref, gb_ref, degp_ref, b_ref, out_ref):
  deg = degp_ref[0, :] + degp_ref[1, :] + 1.0
  dinv = lax.rsqrt(deg)
  ha = p_ref[0, 0] + p_ref[1, 0] + ga_ref[...]
  hb = p_ref[0, 1] + p_ref[1, 1] + gb_ref[...]
  acc = jnp.concatenate([ha, hb], axis=1)
  out_ref[...] = acc * dinv[:, None] + b_ref[0, :]


def kernel(x, edge_index, W, b):
  ei = edge_index.astype(jnp.int32)
  src = ei[0].reshape(NW, N_CHUNKS, CH)
  dst = ei[1].reshape(NW, N_CHUNKS, CH)
  xp = jnp.zeros((N_PAD, D), _f32).at[:N_NODES].set(x)

  deg_part = _deg_kernel(dst)

  ga, gb = pl.pallas_call(
      _prep_body,
      grid=(N_PAD // _RB,),
      in_specs=[
          pl.BlockSpec((_RB, D), lambda i: (i, 0)),
          pl.BlockSpec((D, D), lambda i: (0, 0)),
          pl.BlockSpec((NC, _RB), lambda i: (0, i)),
      ],
      out_specs=[
          pl.BlockSpec((_RB, DH), lambda i: (i, 0)),
          pl.BlockSpec((_RB, DH), lambda i: (i, 0)),
      ],
      out_shape=[
          jax.ShapeDtypeStruct((N_PAD, DH), _f32),
          jax.ShapeDtypeStruct((N_PAD, DH), _f32),
      ],
  )(xp, W, deg_part)

  part = _edge_kernel(src, dst, ga, gb)

  out = pl.pallas_call(
      _combine_body,
      grid=(N_PAD // _RB,),
      in_specs=[
          pl.BlockSpec((NC, 2, _RB, DH), lambda i: (0, 0, i, 0)),
          pl.BlockSpec((_RB, DH), lambda i: (i, 0)),
          pl.BlockSpec((_RB, DH), lambda i: (i, 0)),
          pl.BlockSpec((NC, _RB), lambda i: (0, i)),
          pl.BlockSpec((1, D), lambda i: (0, 0)),
      ],
      out_specs=pl.BlockSpec((_RB, D), lambda i: (i, 0)),
      out_shape=jax.ShapeDtypeStruct((N_PAD, D), _f32),
  )(part, ga, gb, deg_part, b.reshape(1, D))

  return out[:N_NODES]


# speedup vs baseline: 39.1232x; 1.0177x over previous
import functools
import jax
import jax.numpy as jnp
from jax import lax
from jax.experimental import pallas as pl
from jax.experimental.pallas import tpu as pltpu
from jax.experimental.pallas import tpu_sc as plsc

N_NODES = 10000
N_PAD = 10240
N_EDGES = 320000
D = 128
DH = D // 2

NC, NS = 2, 16
NW = NC * NS
E_PER_TILE = N_EDGES // NW
CH = 80
N_CHUNKS = E_PER_TILE // CH
ROWS_PT = N_PAD // NS
NGRP = 15

_mesh = plsc.VectorSubcoreMesh(
    core_axis_name="c", subcore_axis_name="s", num_cores=NC, num_subcores=NS)

_f32 = jnp.float32


def _zero_vmem_2d(ref, rows, cols):
  z = jnp.zeros((16,), _f32)
  per_row = cols // 16

  def body(i, _):
    ref[i // per_row, pl.ds((i % per_row) * 16, 16)] = z
    return 0

  lax.fori_loop(0, rows * per_row, body, 0)


@functools.partial(
    pl.kernel,
    out_type=jax.ShapeDtypeStruct((NC, N_PAD), _f32),
    mesh=_mesh,
    scratch_types=[
        pltpu.VMEM((N_CHUNKS, CH), jnp.int32),
        pltpu.VMEM((CH,), _f32),
        pltpu.VMEM((ROWS_PT,), _f32),
        pltpu.VMEM_SHARED((N_PAD,), _f32),
        pltpu.SemaphoreType.DMA,
    ],
)
def _deg_kernel(dst_hbm, deg_out, idx_v, ones_v, zbuf_v, acc_sh, sem):
  c = lax.axis_index("c")
  s = lax.axis_index("s")
  wid = c * NS + s

  z = jnp.zeros((16,), _f32)
  for j in range(ROWS_PT // 16):
    zbuf_v[pl.ds(j * 16, 16)] = z
  one = jnp.ones((16,), _f32)
  for j in range(CH // 16):
    ones_v[pl.ds(j * 16, 16)] = one
  pltpu.sync_copy(zbuf_v, acc_sh.at[pl.ds(s * ROWS_PT, ROWS_PT)])
  pltpu.sync_copy(dst_hbm.at[wid], idx_v)
  plsc.subcore_barrier()

  W = 16

  def body(t, _):
    pltpu.async_copy(ones_v, acc_sh.at[idx_v.at[t]], sem, add=True)

    @pl.when(t >= W)
    def _():
      pltpu.make_async_copy(ones_v, acc_sh.at[idx_v.at[0]], sem).wait()

    return 0

  lax.fori_loop(0, N_CHUNKS, body, 0)

  def drain(t, _):
    pltpu.make_async_copy(ones_v, acc_sh.at[idx_v.at[0]], sem).wait()
    return 0

  lax.fori_loop(0, W, drain, 0)
  plsc.subcore_barrier()
  pltpu.sync_copy(acc_sh.at[pl.ds(s * ROWS_PT, ROWS_PT)],
                  deg_out.at[c, pl.ds(s * ROWS_PT, ROWS_PT)])


@functools.partial(
    pl.kernel,
    out_type=jax.ShapeDtypeStruct((NC, 2, N_PAD, DH), _f32),
    mesh=_mesh,
    scratch_types=[
        pltpu.VMEM((N_CHUNKS, CH), jnp.int32),
        pltpu.VMEM((N_CHUNKS, CH), jnp.int32),
        [pltpu.VMEM((CH, DH), _f32)] * 8,
        pltpu.VMEM_SHARED((N_PAD, DH), _f32),
        [pltpu.SemaphoreType.DMA] * 8,
    ],
    compiler_params=pltpu.CompilerParams(use_tc_tiling_on_sc=False),
)
def _edge_kernel(src_hbm, dst_hbm, ga_hbm, gb_hbm, part_out, src_v, dst_v,
                 bufs, acc_sh, sems):
  c = lax.axis_index("c")
  s = lax.axis_index("s")
  wid = c * NS + s

  pltpu.sync_copy(src_hbm.at[wid], src_v)
  pltpu.sync_copy(dst_hbm.at[wid], dst_v)
  _zero_vmem_2d(bufs[0], CH, DH)
  for k in range(ROWS_PT // CH):
    pltpu.sync_copy(bufs[0], acc_sh.at[pl.ds(s * ROWS_PT + k * CH, CH), :])
  plsc.subcore_barrier()

  for p, g_hbm in enumerate((ga_hbm, gb_hbm)):

    def fire_gather(t, k):
      pltpu.async_copy(g_hbm.at[src_v.at[t]], bufs[k], sems[k])

    def wait_gather(k):
      pltpu.make_async_copy(g_hbm.at[src_v.at[0]], bufs[k], sems[k]).wait()

    def fire_scatter(t, k):
      pltpu.async_copy(bufs[k], acc_sh.at[dst_v.at[t]], sems[k], add=True)

    def wait_scatter(k):
      pltpu.make_async_copy(bufs[k], acc_sh.at[dst_v.at[0]], sems[k]).wait()

    for k in range(4):
      fire_gather(k, k)

    def body(i, _):
      t0 = i * 8

      @pl.when(i > 0)
      def _():
        for k in range(4):
          wait_scatter(4 + k)
      for k in range(4):
        fire_gather(t0 + 4 + k, 4 + k)
      for k in range(4):
        wait_gather(k)
        fire_scatter(t0 + k, k)

      for k in range(4):
        wait_scatter(k)
      for k in range(4):
        fire_gather(t0 + 8 + k, k)
      for k in range(4):
        wait_gather(4 + k)
        fire_scatter(t0 + 4 + k, 4 + k)
      return 0

    lax.fori_loop(0, NGRP, body, 0)

    wait_scatter(4)
    fire_gather(N_CHUNKS - 1, 4)
    for k in range(4):
      wait_gather(k)
      fire_scatter(120 + k, k)
    wait_gather(4)
    fire_scatter(N_CHUNKS - 1, 4)
    for k in range(4):
      wait_scatter(k)
    for k in range(1, 4):
      wait_scatter(4 + k)
    wait_scatter(4)

    plsc.subcore_barrier()
    pltpu.sync_copy(acc_sh.at[pl.ds(s * ROWS_PT, ROWS_PT), :],
                    part_out.at[c, p, pl.ds(s * ROWS_PT, ROWS_PT), :])
    if p == 0:
      _zero_vmem_2d(bufs[0], CH, DH)
      for k in range(ROWS_PT // CH):
        pltpu.sync_copy(bufs[0], acc_sh.at[pl.ds(s * ROWS_PT + k * CH, CH), :])
      plsc.subcore_barrier()


_RB = 2000


def _dinv_block(degp_ref):
  i = pl.program_id(0)
  deg = degp_ref[0, pl.ds(i, 1), :][0] + degp_ref[1, pl.ds(i, 1), :][0] + 1.0
  return lax.rsqrt(deg)


def _prep_body(x_ref, w_ref, degp_ref, ga_ref, gb_ref):
  h = jnp.dot(x_ref[...], w_ref[...], preferred_element_type=_f32)
  dinv = _dinv_block(degp_ref)
  g = h * dinv[:, None]
  ga_ref[...] = g[:, :DH]
  gb_ref[...] = g[:, DH:]


def _combine_body(p_ref, ga_ref, gb_ref, degp_ref, b_ref, out_ref):
  dinv = _dinv_block(degp_ref)
  ha = p_ref[0, 0] + p_ref[1, 0] + ga_ref[...]
  hb = p_ref[0, 1] + p_ref[1, 1] + gb_ref[...]
  acc = jnp.concatenate([ha, hb], axis=1)
  out_ref[...] = acc * dinv[:, None] + b_ref[0, :]


def kernel(x, edge_index, W, b):
  ei = edge_index.astype(jnp.int32)
  src = ei[0].reshape(NW, N_CHUNKS, CH)
  dst = ei[1].reshape(NW, N_CHUNKS, CH)

  deg_part = _deg_kernel(dst)
  degs = deg_part[:, :N_NODES].reshape(NC, N_NODES // _RB, _RB)

  ga, gb = pl.pallas_call(
      _prep_body,
      grid=(N_NODES // _RB,),
      in_specs=[
          pl.BlockSpec((_RB, D), lambda i: (i, 0)),
          pl.BlockSpec((D, D), lambda i: (0, 0)),
          pl.BlockSpec((NC, N_NODES // _RB, _RB), lambda i: (0, 0, 0)),
      ],
      out_specs=[
          pl.BlockSpec((_RB, DH), lambda i: (i, 0)),
          pl.BlockSpec((_RB, DH), lambda i: (i, 0)),
      ],
      out_shape=[
          jax.ShapeDtypeStruct((N_NODES, DH), _f32),
          jax.ShapeDtypeStruct((N_NODES, DH), _f32),
      ],
  )(x, W, degs)

  part = _edge_kernel(src, dst, ga, gb)

  out = pl.pallas_call(
      _combine_body,
      grid=(N_NODES // _RB,),
      in_specs=[
          pl.BlockSpec((NC, 2, _RB, DH), lambda i: (0, 0, i, 0)),
          pl.BlockSpec((_RB, DH), lambda i: (i, 0)),
          pl.BlockSpec((_RB, DH), lambda i: (i, 0)),
          pl.BlockSpec((NC, N_NODES // _RB, _RB), lambda i: (0, 0, 0)),
          pl.BlockSpec((1, D), lambda i: (0, 0)),
      ],
      out_specs=pl.BlockSpec((_RB, D), lambda i: (i, 0)),
      out_shape=jax.ShapeDtypeStruct((N_NODES, D), _f32),
  )(part, ga, gb, degs, b.reshape(1, D))

  return out


# speedup vs baseline: 43.3580x; 1.1082x over previous
import functools
import jax
import jax.numpy as jnp
from jax import lax
from jax.experimental import pallas as pl
from jax.experimental.pallas import tpu as pltpu
from jax.experimental.pallas import tpu_sc as plsc

N_NODES = 10000
N_PAD = 10240
N_EDGES = 320000
D = 128
DH = D // 2

NC, NS = 2, 16
E_PER_TILE = N_EDGES // NS
CH = 80
N_CHUNKS = E_PER_TILE // CH
ROWS_PT = N_PAD // NS
NGRP = (N_CHUNKS - 2) // 8
DEG_CHUNKS = N_CHUNKS // 2

_mesh = plsc.VectorSubcoreMesh(
    core_axis_name="c", subcore_axis_name="s", num_cores=NC, num_subcores=NS)

_f32 = jnp.float32
_params = pltpu.CompilerParams(use_tc_tiling_on_sc=False)


def _zero_vmem_2d(ref, rows, cols):
  z = jnp.zeros((16,), _f32)
  per_row = cols // 16

  def body(i, _):
    ref[i // per_row, pl.ds((i % per_row) * 16, 16)] = z
    return 0

  lax.fori_loop(0, rows * per_row, body, 0)


@functools.partial(
    pl.kernel,
    out_type=jax.ShapeDtypeStruct((NC, N_PAD), _f32),
    mesh=_mesh,
    scratch_types=[
        pltpu.VMEM((DEG_CHUNKS, CH), jnp.int32),
        pltpu.VMEM((CH,), _f32),
        pltpu.VMEM((ROWS_PT,), _f32),
        pltpu.VMEM_SHARED((N_PAD,), _f32),
        pltpu.SemaphoreType.DMA,
    ],
    compiler_params=_params,
)
def _deg_kernel(dst_hbm, deg_out, idx_v, ones_v, zbuf_v, acc_sh, sem):
  c = lax.axis_index("c")
  s = lax.axis_index("s")
  wid = c * NS + s

  z = jnp.zeros((16,), _f32)
  for j in range(ROWS_PT // 16):
    zbuf_v[pl.ds(j * 16, 16)] = z
  one = jnp.ones((16,), _f32)
  for j in range(CH // 16):
    ones_v[pl.ds(j * 16, 16)] = one
  pltpu.sync_copy(zbuf_v, acc_sh.at[pl.ds(s * ROWS_PT, ROWS_PT)])
  pltpu.sync_copy(
      dst_hbm.at[wid // 2, pl.ds((wid % 2) * DEG_CHUNKS, DEG_CHUNKS), :],
      idx_v)
  plsc.subcore_barrier()

  W = 16

  def body(t, _):
    pltpu.async_copy(ones_v, acc_sh.at[idx_v.at[t]], sem, add=True)

    @pl.when(t >= W)
    def _():
      pltpu.make_async_copy(ones_v, acc_sh.at[idx_v.at[0]], sem).wait()

    return 0

  lax.fori_loop(0, DEG_CHUNKS, body, 0)

  def drain(t, _):
    pltpu.make_async_copy(ones_v, acc_sh.at[idx_v.at[0]], sem).wait()
    return 0

  lax.fori_loop(0, W, drain, 0)
  plsc.subcore_barrier()
  pltpu.sync_copy(acc_sh.at[pl.ds(s * ROWS_PT, ROWS_PT)],
                  deg_out.at[c, pl.ds(s * ROWS_PT, ROWS_PT)])


@functools.partial(
    pl.kernel,
    out_type=jax.ShapeDtypeStruct((NC, N_PAD, DH), _f32),
    mesh=_mesh,
    scratch_types=[
        pltpu.VMEM((N_CHUNKS, CH), jnp.int32),
        pltpu.VMEM((N_CHUNKS, CH), jnp.int32),
        [pltpu.VMEM((CH, DH), _f32)] * 8,
        pltpu.VMEM_SHARED((N_PAD, DH), _f32),
        [pltpu.SemaphoreType.DMA] * 8,
    ],
    compiler_params=_params,
)
def _edge_kernel(src_hbm, dst_hbm, g2_hbm, part_out, src_v, dst_v, bufs,
                 acc_sh, sems):
  c = lax.axis_index("c")
  s = lax.axis_index("s")
  g_hbm = g2_hbm.at[c]

  def fire_gather(t, k):
    pltpu.async_copy(g_hbm.at[src_v.at[t]], bufs[k], sems[k])

  def wait_gather(k):
    pltpu.make_async_copy(g_hbm.at[src_v.at[0]], bufs[k], sems[k]).wait()

  def fire_scatter(t, k):
    pltpu.async_copy(bufs[k], acc_sh.at[dst_v.at[t]], sems[k], add=True)

  def wait_scatter(k):
    pltpu.make_async_copy(bufs[k], acc_sh.at[dst_v.at[0]], sems[k]).wait()

  pltpu.sync_copy(src_hbm.at[s], src_v)
  pltpu.sync_copy(dst_hbm.at[s], dst_v)
  _zero_vmem_2d(bufs[0], CH, DH)
  for k in range(ROWS_PT // CH):
    pltpu.sync_copy(bufs[0], acc_sh.at[pl.ds(s * ROWS_PT + k * CH, CH), :])
  plsc.subcore_barrier()

  for k in range(4):
    fire_gather(k, k)

  def body(i, _):
    t0 = i * 8

    @pl.when(i > 0)
    def _():
      for k in range(4):
        wait_scatter(4 + k)
    for k in range(4):
      fire_gather(t0 + 4 + k, 4 + k)
    for k in range(4):
      wait_gather(k)
      fire_scatter(t0 + k, k)

    for k in range(4):
      wait_scatter(k)
    for k in range(4):
      @pl.when(t0 + 8 + k < N_CHUNKS)
      def _():
        fire_gather(t0 + 8 + k, k)
    for k in range(4):
      wait_gather(4 + k)
      fire_scatter(t0 + 4 + k, 4 + k)
    return 0

  lax.fori_loop(0, NGRP, body, 0)

  for k in range(2):
    wait_gather(k)
    fire_scatter(NGRP * 8 + k, k)
  for k in range(4):
    wait_scatter(4 + k)
  for k in range(2):
    wait_scatter(k)

  plsc.subcore_barrier()
  pltpu.sync_copy(acc_sh.at[pl.ds(s * ROWS_PT, ROWS_PT), :],
                  part_out.at[c, pl.ds(s * ROWS_PT, ROWS_PT), :])


_RB = 2000


def _dinv_block(degp_ref):
  i = pl.program_id(0)
  deg = degp_ref[0, pl.ds(i, 1), :][0] + degp_ref[1, pl.ds(i, 1), :][0] + 1.0
  return lax.rsqrt(deg)


def _prep_body(x_ref, w_ref, degp_ref, g2_ref):
  h = jnp.dot(x_ref[...], w_ref[...], preferred_element_type=_f32)
  dinv = _dinv_block(degp_ref)
  g = h * dinv[:, None]
  g2_ref[0] = g[:, :DH]
  g2_ref[1] = g[:, DH:]


def _combine_body(p_ref, g2_ref, degp_ref, b_ref, out_ref):
  dinv = _dinv_block(degp_ref)
  ha = p_ref[0] + g2_ref[0]
  hb = p_ref[1] + g2_ref[1]
  acc = jnp.concatenate([ha, hb], axis=1)
  out_ref[...] = acc * dinv[:, None] + b_ref[0, :]


def kernel(x, edge_index, W, b):
  ei = edge_index.astype(jnp.int32)
  src = ei[0].reshape(NS, N_CHUNKS, CH)
  dst = ei[1].reshape(NS, N_CHUNKS, CH)

  deg_part = _deg_kernel(dst)
  degs = deg_part[:, :N_NODES].reshape(NC, N_NODES // _RB, _RB)

  g2 = pl.pallas_call(
      _prep_body,
      grid=(N_NODES // _RB,),
      in_specs=[
          pl.BlockSpec((_RB, D), lambda i: (i, 0)),
          pl.BlockSpec((D, D), lambda i: (0, 0)),
          pl.BlockSpec((NC, N_NODES // _RB, _RB), lambda i: (0, 0, 0)),
      ],
      out_specs=pl.BlockSpec((NC, _RB, DH), lambda i: (0, i, 0)),
      out_shape=jax.ShapeDtypeStruct((NC, N_NODES, DH), _f32),
  )(x, W, degs)

  part = _edge_kernel(src, dst, g2)

  out = pl.pallas_call(
      _combine_body,
      grid=(N_NODES // _RB,),
      in_specs=[
          pl.BlockSpec((NC, _RB, DH), lambda i: (0, i, 0)),
          pl.BlockSpec((NC, _RB, DH), lambda i: (0, i, 0)),
          pl.BlockSpec((NC, N_NODES // _RB, _RB), lambda i: (0, 0, 0)),
          pl.BlockSpec((1, D), lambda i: (0, 0)),
      ],
      out_specs=pl.BlockSpec((_RB, D), lambda i: (i, 0)),
      out_shape=jax.ShapeDtypeStruct((N_NODES, D), _f32),
  )(part, g2, degs, b.reshape(1, D))

  return out


# speedup vs baseline: 44.2028x; 1.0195x over previous
import functools
import jax
import jax.numpy as jnp
from jax import lax
from jax.experimental import pallas as pl
from jax.experimental.pallas import tpu as pltpu
from jax.experimental.pallas import tpu_sc as plsc

N_NODES = 10000
N_PAD = 10240
N_EDGES = 320000
D = 128
DH = D // 2

NC, NS = 2, 16
E_PER_TILE = N_EDGES // NS
CH = 80
N_CHUNKS = E_PER_TILE // CH
ROWS_PT = N_PAD // NS
NB = 4
NGRP = (N_CHUNKS - 2) // 8
DEG_CHUNKS = N_CHUNKS // 2

_mesh = plsc.VectorSubcoreMesh(
    core_axis_name="c", subcore_axis_name="s", num_cores=NC, num_subcores=NS)

_f32 = jnp.float32
_params = pltpu.CompilerParams(use_tc_tiling_on_sc=False)


def _zero_vmem_2d(ref, rows, cols):
  z = jnp.zeros((16,), _f32)
  per_row = cols // 16

  def body(i, _):
    ref[i // per_row, pl.ds((i % per_row) * 16, 16)] = z
    return 0

  lax.fori_loop(0, rows * per_row, body, 0)


@functools.partial(
    pl.kernel,
    out_type=jax.ShapeDtypeStruct((NC, N_PAD), _f32),
    mesh=_mesh,
    scratch_types=[
        pltpu.VMEM((DEG_CHUNKS, CH), jnp.int32),
        pltpu.VMEM((CH,), _f32),
        pltpu.VMEM((ROWS_PT,), _f32),
        pltpu.VMEM_SHARED((N_PAD,), _f32),
        pltpu.SemaphoreType.DMA,
    ],
    compiler_params=_params,
)
def _deg_kernel(dst_hbm, deg_out, idx_v, ones_v, zbuf_v, acc_sh, sem):
  c = lax.axis_index("c")
  s = lax.axis_index("s")
  wid = c * NS + s

  z = jnp.zeros((16,), _f32)
  for j in range(ROWS_PT // 16):
    zbuf_v[pl.ds(j * 16, 16)] = z
  one = jnp.ones((16,), _f32)
  for j in range(CH // 16):
    ones_v[pl.ds(j * 16, 16)] = one
  pltpu.sync_copy(zbuf_v, acc_sh.at[pl.ds(s * ROWS_PT, ROWS_PT)])
  pltpu.sync_copy(
      dst_hbm.at[wid // 2, pl.ds((wid % 2) * DEG_CHUNKS, DEG_CHUNKS), :],
      idx_v)
  plsc.subcore_barrier()

  W = 16

  def body(t, _):
    pltpu.async_copy(ones_v, acc_sh.at[idx_v.at[t]], sem, add=True)

    @pl.when(t >= W)
    def _():
      pltpu.make_async_copy(ones_v, acc_sh.at[idx_v.at[0]], sem).wait()

    return 0

  lax.fori_loop(0, DEG_CHUNKS, body, 0)

  def drain(t, _):
    pltpu.make_async_copy(ones_v, acc_sh.at[idx_v.at[0]], sem).wait()
    return 0

  lax.fori_loop(0, W, drain, 0)
  plsc.subcore_barrier()
  pltpu.sync_copy(acc_sh.at[pl.ds(s * ROWS_PT, ROWS_PT)],
                  deg_out.at[c, pl.ds(s * ROWS_PT, ROWS_PT)])


@functools.partial(
    pl.kernel,
    out_type=jax.ShapeDtypeStruct((NC, N_PAD, DH), _f32),
    mesh=_mesh,
    scratch_types=[
        pltpu.VMEM((N_CHUNKS, CH), jnp.int32),
        pltpu.VMEM((N_CHUNKS, CH), jnp.int32),
        [pltpu.VMEM((CH, DH), _f32)] * (2 * NB),
        pltpu.VMEM_SHARED((N_PAD, DH), _f32),
        [pltpu.SemaphoreType.DMA] * (2 * NB),
    ],
    compiler_params=_params,
)
def _edge_kernel(src_hbm, dst_hbm, g2_hbm, part_out, src_v, dst_v, bufs,
                 acc_sh, sems):
  c = lax.axis_index("c")
  s = lax.axis_index("s")
  g_hbm = g2_hbm.at[c]

  def fire_gather(t, k):
    pltpu.async_copy(g_hbm.at[src_v.at[t]], bufs[k], sems[k])

  def wait_gather(k):
    pltpu.make_async_copy(g_hbm.at[src_v.at[0]], bufs[k], sems[k]).wait()

  def fire_scatter(t, k):
    pltpu.async_copy(bufs[k], acc_sh.at[dst_v.at[t]], sems[k], add=True)

  def wait_scatter(k):
    pltpu.make_async_copy(bufs[k], acc_sh.at[dst_v.at[0]], sems[k]).wait()

  pltpu.async_copy(src_hbm.at[s], src_v, sems[0])
  pltpu.async_copy(dst_hbm.at[s], dst_v, sems[1])
  _zero_vmem_2d(bufs[0], CH, DH)
  for k in range(ROWS_PT // CH):
    pltpu.sync_copy(bufs[0], acc_sh.at[pl.ds(s * ROWS_PT + k * CH, CH), :])
  pltpu.make_async_copy(src_hbm.at[s], src_v, sems[0]).wait()
  pltpu.make_async_copy(dst_hbm.at[s], dst_v, sems[1]).wait()
  for k in range(NB):
    fire_gather(k, k)
  plsc.subcore_barrier()

  def body(i, _):
    t0 = i * 2 * NB

    @pl.when(i > 0)
    def _():
      for k in range(NB):
        wait_scatter(NB + k)
    for k in range(NB):
      fire_gather(t0 + NB + k, NB + k)
    for k in range(NB):
      wait_gather(k)
      fire_scatter(t0 + k, k)

    for k in range(NB):
      wait_scatter(k)
    for k in range(NB):
      @pl.when(t0 + 2 * NB + k < N_CHUNKS)
      def _():
        fire_gather(t0 + 2 * NB + k, k)
    for k in range(NB):
      wait_gather(NB + k)
      fire_scatter(t0 + NB + k, NB + k)
    return 0

  lax.fori_loop(0, NGRP, body, 0)

  for k in range(2):
    wait_gather(k)
    fire_scatter(NGRP * 2 * NB + k, k)
  for k in range(NB):
    wait_scatter(NB + k)
  for k in range(2):
    wait_scatter(k)

  plsc.subcore_barrier()
  pltpu.sync_copy(acc_sh.at[pl.ds(s * ROWS_PT, ROWS_PT), :],
                  part_out.at[c, pl.ds(s * ROWS_PT, ROWS_PT), :])


_RB = 2000


def _dinv_block(degp_ref):
  i = pl.program_id(0)
  deg = degp_ref[0, pl.ds(i, 1), :][0] + degp_ref[1, pl.ds(i, 1), :][0] + 1.0
  return lax.rsqrt(deg)


def _prep_body(x_ref, w_ref, degp_ref, g2_ref):
  h = jnp.dot(x_ref[...], w_ref[...], preferred_element_type=_f32)
  dinv = _dinv_block(degp_ref)
  g = h * dinv[:, None]
  g2_ref[0] = g[:, :DH]
  g2_ref[1] = g[:, DH:]


def _combine_body(p_ref, g2_ref, degp_ref, b_ref, out_ref):
  dinv = _dinv_block(degp_ref)
  ha = p_ref[0] + g2_ref[0]
  hb = p_ref[1] + g2_ref[1]
  acc = jnp.concatenate([ha, hb], axis=1)
  out_ref[...] = acc * dinv[:, None] + b_ref[0, :]


def kernel(x, edge_index, W, b):
  ei = edge_index.astype(jnp.int32)
  src = ei[0].reshape(NS, N_CHUNKS, CH)
  dst = ei[1].reshape(NS, N_CHUNKS, CH)

  deg_part = _deg_kernel(dst)
  degs = deg_part[:, :N_NODES].reshape(NC, N_NODES // _RB, _RB)

  g2 = pl.pallas_call(
      _prep_body,
      grid=(N_NODES // _RB,),
      in_specs=[
          pl.BlockSpec((_RB, D), lambda i: (i, 0)),
          pl.BlockSpec((D, D), lambda i: (0, 0)),
          pl.BlockSpec((NC, N_NODES // _RB, _RB), lambda i: (0, 0, 0)),
      ],
      out_specs=pl.BlockSpec((NC, _RB, DH), lambda i: (0, i, 0)),
      out_shape=jax.ShapeDtypeStruct((NC, N_NODES, DH), _f32),
  )(x, W, degs)

  part = _edge_kernel(src, dst, g2)

  out = pl.pallas_call(
      _combine_body,
      grid=(N_NODES // _RB,),
      in_specs=[
          pl.BlockSpec((NC, _RB, DH), lambda i: (0, i, 0)),
          pl.BlockSpec((NC, _RB, DH), lambda i: (0, i, 0)),
          pl.BlockSpec((NC, N_NODES // _RB, _RB), lambda i: (0, 0, 0)),
          pl.BlockSpec((1, D), lambda i: (0, 0)),
      ],
      out_specs=pl.BlockSpec((_RB, D), lambda i: (i, 0)),
      out_shape=jax.ShapeDtypeStruct((N_NODES, D), _f32),
  )(part, g2, degs, b.reshape(1, D))

  return out


# speedup vs baseline: 44.3197x; 1.0026x over previous
import functools
import jax
import jax.numpy as jnp
from jax import lax
from jax.experimental import pallas as pl
from jax.experimental.pallas import tpu as pltpu
from jax.experimental.pallas import tpu_sc as plsc

N_NODES = 10000
N_PAD = 10240
N_EDGES = 320000
D = 128
DH = D // 2

NC, NS = 2, 16
E_PER_TILE = N_EDGES // NS
CH = 80
N_CHUNKS = E_PER_TILE // CH
ROWS_PT = N_PAD // NS
NB = 4
NGRP = (N_CHUNKS - 2) // 8
DEG_CHUNKS = N_CHUNKS // 2

_mesh = plsc.VectorSubcoreMesh(
    core_axis_name="c", subcore_axis_name="s", num_cores=NC, num_subcores=NS)

_f32 = jnp.float32
_params = pltpu.CompilerParams(use_tc_tiling_on_sc=False)


def _zero_vmem_2d(ref, rows, cols):
  z = jnp.zeros((16,), _f32)
  per_row = cols // 16

  def body(i, _):
    ref[i // per_row, pl.ds((i % per_row) * 16, 16)] = z
    return 0

  lax.fori_loop(0, rows * per_row, body, 0)


@functools.partial(
    pl.kernel,
    out_type=jax.ShapeDtypeStruct((NC, N_PAD), _f32),
    mesh=_mesh,
    scratch_types=[
        pltpu.VMEM((DEG_CHUNKS, CH), jnp.int32),
        pltpu.VMEM((CH,), _f32),
        pltpu.VMEM((ROWS_PT,), _f32),
        pltpu.VMEM_SHARED((N_PAD,), _f32),
        pltpu.SemaphoreType.DMA,
    ],
    compiler_params=_params,
)
def _deg_kernel(dst_hbm, deg_out, idx_v, ones_v, zbuf_v, acc_sh, sem):
  c = lax.axis_index("c")
  s = lax.axis_index("s")
  wid = c * NS + s

  z = jnp.zeros((16,), _f32)
  for j in range(ROWS_PT // 16):
    zbuf_v[pl.ds(j * 16, 16)] = z
  one = jnp.ones((16,), _f32)
  for j in range(CH // 16):
    ones_v[pl.ds(j * 16, 16)] = one
  pltpu.sync_copy(zbuf_v, acc_sh.at[pl.ds(s * ROWS_PT, ROWS_PT)])
  pltpu.sync_copy(
      dst_hbm.at[wid // 2, pl.ds((wid % 2) * DEG_CHUNKS, DEG_CHUNKS), :],
      idx_v)
  plsc.subcore_barrier()

  W = 32

  def body(t, _):
    pltpu.async_copy(ones_v, acc_sh.at[idx_v.at[t]], sem, add=True)

    @pl.when(t >= W)
    def _():
      pltpu.make_async_copy(ones_v, acc_sh.at[idx_v.at[0]], sem).wait()

    return 0

  lax.fori_loop(0, DEG_CHUNKS, body, 0)

  def drain(t, _):
    pltpu.make_async_copy(ones_v, acc_sh.at[idx_v.at[0]], sem).wait()
    return 0

  lax.fori_loop(0, W, drain, 0)
  plsc.subcore_barrier()
  pltpu.sync_copy(acc_sh.at[pl.ds(s * ROWS_PT, ROWS_PT)],
                  deg_out.at[c, pl.ds(s * ROWS_PT, ROWS_PT)])


@functools.partial(
    pl.kernel,
    out_type=jax.ShapeDtypeStruct((NC, N_PAD, DH), _f32),
    mesh=_mesh,
    scratch_types=[
        pltpu.VMEM((N_CHUNKS, CH), jnp.int32),
        pltpu.VMEM((N_CHUNKS, CH), jnp.int32),
        [pltpu.VMEM((CH, DH), _f32)] * (2 * NB),
        pltpu.VMEM_SHARED((N_PAD, DH), _f32),
        [pltpu.SemaphoreType.DMA] * (2 * NB),
    ],
    compiler_params=_params,
)
def _edge_kernel(src_hbm, dst_hbm, g2_hbm, part_out, src_v, dst_v, bufs,
                 acc_sh, sems):
  c = lax.axis_index("c")
  s = lax.axis_index("s")
  g_hbm = g2_hbm.at[c]

  def fire_gather(t, k):
    pltpu.async_copy(g_hbm.at[src_v.at[t]], bufs[k], sems[k])

  def wait_gather(k):
    pltpu.make_async_copy(g_hbm.at[src_v.at[0]], bufs[k], sems[k]).wait()

  def fire_scatter(t, k):
    pltpu.async_copy(bufs[k], acc_sh.at[dst_v.at[t]], sems[k], add=True)

  def wait_scatter(k):
    pltpu.make_async_copy(bufs[k], acc_sh.at[dst_v.at[0]], sems[k]).wait()

  pltpu.async_copy(src_hbm.at[s], src_v, sems[0])
  pltpu.async_copy(dst_hbm.at[s], dst_v, sems[1])
  _zero_vmem_2d(bufs[0], CH, DH)
  for k in range(ROWS_PT // CH):
    pltpu.async_copy(bufs[0], acc_sh.at[pl.ds(s * ROWS_PT + k * CH, CH), :],
                     sems[2])
  for k in range(ROWS_PT // CH):
    pltpu.make_async_copy(bufs[0],
                          acc_sh.at[pl.ds(s * ROWS_PT, CH), :], sems[2]).wait()
  pltpu.make_async_copy(src_hbm.at[s], src_v, sems[0]).wait()
  pltpu.make_async_copy(dst_hbm.at[s], dst_v, sems[1]).wait()
  for k in range(NB):
    fire_gather(k, k)
  plsc.subcore_barrier()

  def body(i, _):
    t0 = i * 2 * NB

    @pl.when(i > 0)
    def _():
      for k in range(NB):
        wait_scatter(NB + k)
    for k in range(NB):
      fire_gather(t0 + NB + k, NB + k)
    for k in range(NB):
      wait_gather(k)
      fire_scatter(t0 + k, k)

    for k in range(NB):
      wait_scatter(k)
    for k in range(NB):
      @pl.when(t0 + 2 * NB + k < N_CHUNKS)
      def _():
        fire_gather(t0 + 2 * NB + k, k)
    for k in range(NB):
      wait_gather(NB + k)
      fire_scatter(t0 + NB + k, NB + k)
    return 0

  lax.fori_loop(0, NGRP, body, 0)

  for k in range(2):
    wait_gather(k)
    fire_scatter(NGRP * 2 * NB + k, k)
  for k in range(NB):
    wait_scatter(NB + k)
  for k in range(2):
    wait_scatter(k)

  plsc.subcore_barrier()
  pltpu.sync_copy(acc_sh.at[pl.ds(s * ROWS_PT, ROWS_PT), :],
                  part_out.at[c, pl.ds(s * ROWS_PT, ROWS_PT), :])


_RB = 2000


def _dinv_block(degp_ref):
  i = pl.program_id(0)
  deg = degp_ref[0, pl.ds(i, 1), :][0] + degp_ref[1, pl.ds(i, 1), :][0] + 1.0
  return lax.rsqrt(deg)


def _prep_body(x_ref, w_ref, degp_ref, g2_ref):
  h = jnp.dot(x_ref[...], w_ref[...], preferred_element_type=_f32)
  dinv = _dinv_block(degp_ref)
  g = h * dinv[:, None]
  g2_ref[0] = g[:, :DH]
  g2_ref[1] = g[:, DH:]


def _combine_body(p_ref, g2_ref, degp_ref, b_ref, out_ref):
  dinv = _dinv_block(degp_ref)
  ha = p_ref[0] + g2_ref[0]
  hb = p_ref[1] + g2_ref[1]
  acc = jnp.concatenate([ha, hb], axis=1)
  out_ref[...] = acc * dinv[:, None] + b_ref[0, :]


def kernel(x, edge_index, W, b):
  ei = edge_index.astype(jnp.int32)
  src = ei[0].reshape(NS, N_CHUNKS, CH)
  dst = ei[1].reshape(NS, N_CHUNKS, CH)

  deg_part = _deg_kernel(dst)
  degs = deg_part[:, :N_NODES].reshape(NC, N_NODES // _RB, _RB)

  g2 = pl.pallas_call(
      _prep_body,
      grid=(N_NODES // _RB,),
      in_specs=[
          pl.BlockSpec((_RB, D), lambda i: (i, 0)),
          pl.BlockSpec((D, D), lambda i: (0, 0)),
          pl.BlockSpec((NC, N_NODES // _RB, _RB), lambda i: (0, 0, 0)),
      ],
      out_specs=pl.BlockSpec((NC, _RB, DH), lambda i: (0, i, 0)),
      out_shape=jax.ShapeDtypeStruct((NC, N_NODES, DH), _f32),
  )(x, W, degs)

  part = _edge_kernel(src, dst, g2)

  out = pl.pallas_call(
      _combine_body,
      grid=(N_NODES // _RB,),
      in_specs=[
          pl.BlockSpec((NC, _RB, DH), lambda i: (0, i, 0)),
          pl.BlockSpec((NC, _RB, DH), lambda i: (0, i, 0)),
          pl.BlockSpec((NC, N_NODES // _RB, _RB), lambda i: (0, 0, 0)),
          pl.BlockSpec((1, D), lambda i: (0, 0)),
      ],
      out_specs=pl.BlockSpec((_RB, D), lambda i: (i, 0)),
      out_shape=jax.ShapeDtypeStruct((N_NODES, D), _f32),
  )(part, g2, degs, b.reshape(1, D))

  return out
